# Initial kernel scaffold; baseline (speedup 1.0000x reference)
import jax
import jax.numpy as jnp
from jax.experimental import pallas as pl


def kernel(x, base_bias, bias_0, bias_1, ctx_maps_0, ctx_bias_0, weights_0, ctx_maps_1, ctx_bias_1, weights_1, ctx_maps_2, ctx_bias_2, weights_2):
    raise NotImplementedError("write your pallas kernel here")


# speedup vs baseline: 94.8590x; 94.8590x over previous
import math

import jax
import jax.numpy as jnp
from jax.experimental import pallas as pl
from jax.experimental.pallas import tpu as pltpu

_PRED_CLIP = 0.001
_LO = math.log(_PRED_CLIP / (1.0 - _PRED_CLIP))
_HI = math.log((1.0 - _PRED_CLIP) / _PRED_CLIP)
_BB = 256


def _prep_layer(cm, cb, w, S, shift, P):
    s = cm.shape[1]
    pf, pb = shift, S - s - shift
    cmt = jnp.pad(jnp.transpose(cm[0], (1, 0, 2)), ((0, 0), (pf, pb), (0, 0)))
    cmT = jnp.transpose(cmt.reshape(4 * S, cm.shape[3]), (1, 0))
    cbt = jnp.pad(jnp.transpose(cb[0, :, :, 0], (1, 0)), ((0, 0), (pf, pb)),
                  constant_values=jnp.inf)
    cbp = cbt.reshape(1, 4 * S)
    wp = jnp.pad(jnp.transpose(w[0], (1, 0, 2)),
                 ((0, 0), (pf, pb), (0, P - w.shape[3])))
    wT = jnp.transpose(wp.reshape(16 * S, P), (1, 0))
    return cmT, cbp, wT


def _ctx_idx(x, cmT, cbp, S):
    d = jnp.dot(x, cmT, preferred_element_type=jnp.float32)
    bits = (d > cbp).astype(jnp.float32)
    return (bits[:, 0 * S:1 * S] + 2.0 * bits[:, 1 * S:2 * S]
            + 4.0 * bits[:, 2 * S:3 * S] + 8.0 * bits[:, 3 * S:4 * S])


def _select16(a, idx, S):
    out = jnp.where(idx == 0.0, a[:, 0:S], 0.0)
    for k in range(1, 16):
        out = out + jnp.where(idx == float(k), a[:, k * S:(k + 1) * S], 0.0)
    return out


def _gln_body(x_ref, sc_ref, cm0_ref, cb0_ref, w0_ref, cm1_ref, cb1_ref,
              w1_ref, cm2_ref, cb2_ref, w2_ref, o_ref):
    x = x_ref[...]
    lane256 = jax.lax.broadcasted_iota(jnp.int32, (1, 256), 1)
    lane128 = jax.lax.broadcasted_iota(jnp.int32, (1, 128), 1)

    xc = jnp.clip(x, _PRED_CLIP, 1.0 - _PRED_CLIP)
    l0 = jnp.log(xc / (1.0 - xc))
    l0 = jnp.where(lane256 == 0, sc_ref[0], l0)

    idx0 = _ctx_idx(x, cm0_ref[...], cb0_ref[...], 128)
    a0 = jnp.dot(l0, w0_ref[...], preferred_element_type=jnp.float32)
    out0 = _select16(a0, idx0, 128)
    l1 = jnp.where(lane128 == 0, sc_ref[1], jnp.clip(out0, _LO, _HI))

    idx1 = _ctx_idx(x, cm1_ref[...], cb1_ref[...], 128)
    a1 = jnp.dot(l1, w1_ref[...], preferred_element_type=jnp.float32)
    out1 = _select16(a1, idx1, 128)
    l2 = jnp.where(lane128 == 0, sc_ref[2], jnp.clip(out1, _LO, _HI))

    idx2 = _ctx_idx(x, cm2_ref[...], cb2_ref[...], 8)
    a2 = jnp.dot(l2, w2_ref[...], preferred_element_type=jnp.float32)
    out2 = _select16(a2, idx2, 8)
    o_ref[...] = jax.nn.sigmoid(jnp.clip(out2[:, 0:1], _LO, _HI))


def kernel(x, base_bias, bias_0, bias_1, ctx_maps_0, ctx_bias_0, weights_0,
           ctx_maps_1, ctx_bias_1, weights_1, ctx_maps_2, ctx_bias_2,
           weights_2):
    B = x.shape[0]
    cm0T, cb0, w0T = _prep_layer(ctx_maps_0, ctx_bias_0, weights_0, 128, 1, 256)
    cm1T, cb1, w1T = _prep_layer(ctx_maps_1, ctx_bias_1, weights_1, 128, 1, 128)
    cm2T, cb2, w2T = _prep_layer(ctx_maps_2, ctx_bias_2, weights_2, 8, 0, 128)
    scalars = jnp.stack([base_bias, bias_0[0, 0, 0], bias_1[0, 0, 0]])

    rep = lambda i: (0, 0)
    grid = (B // _BB,)
    probs = pl.pallas_call(
        _gln_body,
        grid=grid,
        in_specs=[
            pl.BlockSpec((_BB, 256), lambda i: (i, 0)),
            pl.BlockSpec(memory_space=pltpu.SMEM),
            pl.BlockSpec((256, 512), rep),
            pl.BlockSpec((1, 512), rep),
            pl.BlockSpec((256, 2048), rep),
            pl.BlockSpec((256, 512), rep),
            pl.BlockSpec((1, 512), rep),
            pl.BlockSpec((128, 2048), rep),
            pl.BlockSpec((256, 32), rep),
            pl.BlockSpec((1, 32), rep),
            pl.BlockSpec((128, 128), rep),
        ],
        out_specs=pl.BlockSpec((_BB, 1), lambda i: (i, 0)),
        out_shape=jax.ShapeDtypeStruct((B, 1), jnp.float32),
    )(x, scalars, cm0T, cb0, w0T, cm1T, cb1, w1T, cm2T, cb2, w2T)
    return probs
